# Initial kernel scaffold; baseline (speedup 1.0000x reference)
import jax
import jax.numpy as jnp
from jax.experimental import pallas as pl


def kernel(input, weight):
    raise NotImplementedError("write your pallas kernel here")


# speedup vs baseline: 2.8560x; 2.8560x over previous
import functools

import jax
import jax.numpy as jnp
from jax import lax
from jax.experimental import pallas as pl
from jax.experimental.pallas import tpu as pltpu
from jax.experimental.pallas import tpu_sc as plsc

D = 32
B = 16384
BAG = 50
NC = 2
NS = 16
NW = NC * NS
BPW = B // NW

_mesh = plsc.VectorSubcoreMesh(core_axis_name="c", subcore_axis_name="s")


@functools.partial(
    pl.kernel,
    mesh=_mesh,
    out_type=jax.ShapeDtypeStruct((B, D), jnp.float32),
    scratch_types=[
        pltpu.VMEM((BAG, BPW), jnp.int32),
        pltpu.VMEM((BPW, D), jnp.float32),
        pltpu.SemaphoreType.DMA,
    ],
    compiler_params=pltpu.CompilerParams(use_tc_tiling_on_sc=False),
)
def _bag(idx_hbm, w_hbm, out_hbm, idx_v, acc, sem):
    wid = lax.axis_index("s") * NC + lax.axis_index("c")
    pltpu.sync_copy(idx_hbm.at[wid], idx_v)
    pltpu.async_copy(w_hbm.at[idx_v.at[0]], acc, sem).wait()
    for j in range(1, BAG):
        pltpu.async_copy(w_hbm.at[idx_v.at[j]], acc, sem, add=True).wait()
    pltpu.sync_copy(acc, out_hbm.at[pl.ds(wid * BPW, BPW)])


def kernel(input, weight):
    idx = input.astype(jnp.int32)
    idx_r = idx.reshape(NW, BPW, BAG).transpose(0, 2, 1)
    return _bag(idx_r, weight)


# speedup vs baseline: 3.0377x; 1.0636x over previous
import functools

import jax
import jax.numpy as jnp
from jax import lax
from jax.experimental import pallas as pl
from jax.experimental.pallas import tpu as pltpu
from jax.experimental.pallas import tpu_sc as plsc

D = 32
B = 16384
BAG = 50
NC = 2
NS = 16
NW = NC * NS
BPW = B // NW

_mesh = plsc.VectorSubcoreMesh(core_axis_name="c", subcore_axis_name="s")


@functools.partial(
    pl.kernel,
    mesh=_mesh,
    out_type=jax.ShapeDtypeStruct((B, D), jnp.float32),
    scratch_types=[
        pltpu.VMEM((BAG, BPW), jnp.int32),
        pltpu.VMEM((BPW, D), jnp.float32),
        pltpu.SemaphoreType.DMA,
    ],
    compiler_params=pltpu.CompilerParams(use_tc_tiling_on_sc=False),
)
def _bag(idx_hbm, w_hbm, out_hbm, idx_v, acc, sem):
    wid = lax.axis_index("s") * NC + lax.axis_index("c")
    idx_cp = pltpu.async_copy(idx_hbm.at[wid], idx_v, sem)
    zero = jnp.zeros((16,), jnp.float32)

    def zbody(i, carry):
        acc[i, pl.ds(0, 16)] = zero
        acc[i, pl.ds(16, 16)] = zero
        return carry

    lax.fori_loop(0, BPW, zbody, 0)
    idx_cp.wait()
    descs = [
        pltpu.async_copy(w_hbm.at[idx_v.at[j]], acc, sem, add=True)
        for j in range(BAG)
    ]
    for d in descs:
        d.wait()
    pltpu.sync_copy(acc, out_hbm.at[pl.ds(wid * BPW, BPW)])


def kernel(input, weight):
    idx = input.astype(jnp.int32)
    idx_r = idx.reshape(NW, BPW, BAG).transpose(0, 2, 1)
    return _bag(idx_r, weight)
